# baseline (speedup 1.0000x reference)
import functools

import jax
import jax.numpy as jnp
from jax import lax
from jax.experimental import pallas as pl
from jax.experimental.pallas import tpu as pltpu
from jax.experimental.pallas import tpu_sc as plsc

_NUM_BOXES = 100000
_DIM = 64
_ROW = 2 * _DIM
_BATCH = 16384

_INFO = plsc.get_sparse_core_info()
_NC = _INFO.num_cores
_NS = _INFO.num_subcores
_NW = _NC * _NS
_B_PER_W = _BATCH // _NW
_NCHUNK = 4
_CH = _B_PER_W // _NCHUNK
_L = 16


@functools.partial(
    pl.kernel,
    out_type=jax.ShapeDtypeStruct((1, _BATCH, 2, _DIM), jnp.float32),
    mesh=plsc.VectorSubcoreMesh(core_axis_name="c", subcore_axis_name="s"),
    scratch_types=[
        pltpu.VMEM((_B_PER_W,), jnp.int32),
        pltpu.VMEM((2, _CH, _ROW), jnp.float32),
        pltpu.VMEM((2, _CH, 2, _DIM), jnp.float32),
        pltpu.SemaphoreType.DMA,
        pltpu.SemaphoreType.DMA,
        pltpu.SemaphoreType.DMA,
        pltpu.SemaphoreType.DMA,
    ],
)
def _gather_rows(table_hbm, ids_hbm, out_hbm, idx_v, rows, padded, g0, g1, s0, s1):
    wid = lax.axis_index("s") * _NC + lax.axis_index("c")
    base = wid * _B_PER_W
    gsems = (g0, g1)
    ssems = (s0, s1)
    pltpu.sync_copy(ids_hbm.at[pl.ds(base, _B_PER_W)], idx_v)

    def start_gather(c):
        return pltpu.async_copy(
            table_hbm.at[idx_v.at[pl.ds(c * _CH, _CH)]],
            rows.at[c % 2],
            gsems[c % 2],
        )

    def start_store(c):
        return pltpu.async_copy(
            padded.at[c % 2],
            out_hbm.at[0].at[pl.ds(base + c * _CH, _CH)],
            ssems[c % 2],
        )

    def shuffle(c):
        src = rows.at[c % 2]
        dst = padded.at[c % 2]

        def body(b, carry):
            for r in range(2):
                for k in range(_DIM // _L):
                    dst[b, r, pl.ds(k * _L, _L)] = src[b, pl.ds(r * _DIM + k * _L, _L)]
            return carry

        lax.fori_loop(0, _CH, body, 0)

    gathers = [None] * _NCHUNK
    stores = [None] * _NCHUNK
    gathers[0] = start_gather(0)
    gathers[1] = start_gather(1)
    for c in range(_NCHUNK):
        gathers[c].wait()
        if c >= 2:
            stores[c - 2].wait()
        shuffle(c)
        stores[c] = start_store(c)
        if c + 2 < _NCHUNK:
            gathers[c + 2] = start_gather(c + 2)
    stores[_NCHUNK - 2].wait()
    stores[_NCHUNK - 1].wait()


def kernel(boxes, ids):
    num_models, num_boxes, two, dim = boxes.shape
    table = boxes.reshape(num_boxes, two * dim)
    return _gather_rows(table, ids.astype(jnp.int32))


# speedup vs baseline: 1.1315x; 1.1315x over previous
import functools

import jax
import jax.numpy as jnp
from jax import lax
from jax.experimental import pallas as pl
from jax.experimental.pallas import tpu as pltpu
from jax.experimental.pallas import tpu_sc as plsc

_NUM_BOXES = 100000
_ROW = 128
_BATCH = 16384

_INFO = plsc.get_sparse_core_info()
_NC = _INFO.num_cores
_NS = _INFO.num_subcores
_NW = _NC * _NS
_BPW = _BATCH // _NW
_IC = 128
_CH = _BPW // _IC


@functools.partial(
    pl.kernel,
    out_type=jax.ShapeDtypeStruct((_NW, _CH, _IC, _ROW), jnp.float32),
    mesh=plsc.VectorSubcoreMesh(core_axis_name="c", subcore_axis_name="s"),
    compiler_params=pltpu.CompilerParams(needs_layout_passes=False),
    scratch_types=[
        pltpu.VMEM((_CH, _IC), jnp.int32),
        pltpu.VMEM((_CH, _IC, _ROW), jnp.float32),
        pltpu.SemaphoreType.DMA,
    ],
)
def _gather_rows(table_hbm, idx_hbm, out_hbm, idx_v, rows_v, sem):
    wid = lax.axis_index("s") * _NC + lax.axis_index("c")
    pltpu.sync_copy(idx_hbm.at[wid], idx_v)
    copies = [
        pltpu.async_copy(table_hbm.at[idx_v.at[j]], rows_v.at[j], sem)
        for j in range(_CH)
    ]
    for c in copies:
        c.wait()
    pltpu.sync_copy(rows_v, out_hbm.at[wid])


def kernel(boxes, ids):
    num_models, num_boxes, two, dim = boxes.shape
    table = boxes.reshape(num_boxes, two * dim)
    idx = ids.astype(jnp.int32).reshape(_NW, _CH, _IC)
    out = _gather_rows(table, idx)
    return out.reshape(num_models, _BATCH, two, dim)
